# baseline (speedup 1.0000x reference)
import jax
import jax.numpy as jnp
from jax.experimental import pallas as pl

BN_EPS = 1e-5
COS_EPS = 1e-8
BLOCK_E = 4096


def _pae_block(x_ref, w1_ref, b1_ref, w2_ref, b2_ref, o_ref):
    x = x_ref[...]
    w1 = w1_ref[...]
    b1 = b1_ref[...]
    w2 = w2_ref[...]
    b2 = b2_ref[...]
    in_dim = w1.shape[0]
    x1 = x[:, :in_dim].astype(jnp.bfloat16)
    x2 = x[:, in_dim:].astype(jnp.bfloat16)
    w1b = w1.astype(jnp.bfloat16)
    w2b = w2.astype(jnp.bfloat16)
    a1 = jnp.maximum(jnp.dot(x1, w1b, preferred_element_type=jnp.float32) + b1, 0.0)
    a2 = jnp.maximum(jnp.dot(x2, w1b, preferred_element_type=jnp.float32) + b1, 0.0)
    h1 = jnp.dot(a1.astype(jnp.bfloat16), w2b, preferred_element_type=jnp.float32) + b2
    h2 = jnp.dot(a2.astype(jnp.bfloat16), w2b, preferred_element_type=jnp.float32) + b2
    s11 = jnp.sum(h1 * h1, axis=1)
    s22 = jnp.sum(h2 * h2, axis=1)
    s12 = jnp.sum(h1 * h2, axis=1)
    n1 = jnp.maximum(jnp.sqrt(s11), COS_EPS)
    n2 = jnp.maximum(jnp.sqrt(s22), COS_EPS)
    o_ref[...] = (s12 / (n1 * n2) + 1.0) * 0.5


def kernel(edge_index, edgenet_input, flag, W1, b1, gamma, beta,
           running_mean, running_var, W2, b2):
    n_edges, feat = edgenet_input.shape
    in_dim = feat // 2
    hidden = W1.shape[1]

    scale = gamma * jax.lax.rsqrt(running_var + BN_EPS)
    w2f = W2 * scale[:, None]
    b2f = b2 + (beta - running_mean * scale) @ W2

    edge_weight = pl.pallas_call(
        _pae_block,
        grid=(pl.cdiv(n_edges, BLOCK_E),),
        in_specs=[
            pl.BlockSpec((BLOCK_E, feat), lambda i: (i, 0)),
            pl.BlockSpec((in_dim, hidden), lambda i: (0, 0)),
            pl.BlockSpec((1, hidden), lambda i: (0, 0)),
            pl.BlockSpec((hidden, hidden), lambda i: (0, 0)),
            pl.BlockSpec((1, hidden), lambda i: (0, 0)),
        ],
        out_specs=pl.BlockSpec((BLOCK_E,), lambda i: (i,)),
        out_shape=jax.ShapeDtypeStruct((n_edges,), jnp.float32),
    )(edgenet_input, W1, b1.reshape(1, hidden), w2f, b2f.reshape(1, hidden))

    return edge_weight, edge_index


# speedup vs baseline: 3.3726x; 3.3726x over previous
import jax
import jax.numpy as jnp
from jax.experimental import pallas as pl

BN_EPS = 1e-5
COS_EPS = 1e-8
BLOCK_E = 4096


def _pae_block(xt_ref, w1t_ref, b1t_ref, w2t_ref, b2t_ref, o_ref):
    xt = xt_ref[...]
    w1t = w1t_ref[...]
    b1t = b1t_ref[...]
    w2t = w2t_ref[...]
    b2t = b2t_ref[...]
    in_dim = w1t.shape[1]
    x1t = xt[:in_dim, :]
    x2t = xt[in_dim:, :]
    a1 = jnp.maximum(jnp.dot(w1t, x1t, preferred_element_type=jnp.float32) + b1t, 0.0)
    a2 = jnp.maximum(jnp.dot(w1t, x2t, preferred_element_type=jnp.float32) + b1t, 0.0)
    h1 = jnp.dot(w2t, a1.astype(jnp.bfloat16), preferred_element_type=jnp.float32) + b2t
    h2 = jnp.dot(w2t, a2.astype(jnp.bfloat16), preferred_element_type=jnp.float32) + b2t
    s11 = jnp.sum(h1 * h1, axis=0)
    s22 = jnp.sum(h2 * h2, axis=0)
    s12 = jnp.sum(h1 * h2, axis=0)
    n1 = jnp.maximum(jnp.sqrt(s11), COS_EPS)
    n2 = jnp.maximum(jnp.sqrt(s22), COS_EPS)
    o_ref[...] = (s12 / (n1 * n2) + 1.0) * 0.5


def kernel(edge_index, edgenet_input, flag, W1, b1, gamma, beta,
           running_mean, running_var, W2, b2):
    n_edges, feat = edgenet_input.shape
    in_dim = feat // 2
    hidden = W1.shape[1]

    xt = edgenet_input.T.astype(jnp.bfloat16)
    scale = gamma * jax.lax.rsqrt(running_var + BN_EPS)
    w1t = W1.T.astype(jnp.bfloat16)
    w2t = (W2 * scale[:, None]).T.astype(jnp.bfloat16)
    b2f = b2 + (beta - running_mean * scale) @ W2

    edge_weight = pl.pallas_call(
        _pae_block,
        grid=(pl.cdiv(n_edges, BLOCK_E),),
        in_specs=[
            pl.BlockSpec((feat, BLOCK_E), lambda i: (0, i)),
            pl.BlockSpec((hidden, in_dim), lambda i: (0, 0)),
            pl.BlockSpec((hidden, 1), lambda i: (0, 0)),
            pl.BlockSpec((hidden, hidden), lambda i: (0, 0)),
            pl.BlockSpec((hidden, 1), lambda i: (0, 0)),
        ],
        out_specs=pl.BlockSpec((BLOCK_E,), lambda i: (i,)),
        out_shape=jax.ShapeDtypeStruct((n_edges,), jnp.float32),
    )(xt, w1t, b1.reshape(hidden, 1), w2t, b2f.reshape(hidden, 1))

    return edge_weight, edge_index
